# Initial kernel scaffold; baseline (speedup 1.0000x reference)
import jax
import jax.numpy as jnp
from jax.experimental import pallas as pl


def kernel(g_x, next_demand, a, edge_index, reverse_edge_index, Wl3, bl3, Wr3, br3, att3, bias3, Wl4, bl4, Wr4, br4, att4, bias4, Wl1, bl1, Wr1, br1, att1, bias1, Wl2, bl2, Wr2, br2, att2, bias2, fc_s_W, fc_s_b, fc_a_W, fc_a_b, fc_cat_W, fc_cat_b, fc_out_W, fc_out_b):
    raise NotImplementedError("write your pallas kernel here")


# speedup vs baseline: 7.7816x; 7.7816x over previous
import functools

import jax
import jax.numpy as jnp
from jax import lax
from jax.experimental import pallas as pl
from jax.experimental.pallas import tpu as pltpu

N_NODE = 1000
N_EDGE = 4000
N_GOODS = 4
B = 64
N = B * N_NODE
E = B * N_EDGE
IN_CH = 16
HID = 32
HEADS = 4
OUT_CH = 16
H1 = 1024
EP = E + N


def _mm_kernel(x_ref, w_ref, b_ref, o_ref):
    o_ref[...] = jnp.dot(x_ref[...], w_ref[...],
                         preferred_element_type=jnp.float32) + b_ref[...]


def _node_mm(x, W, b, block_m=4000):
    n, k = x.shape
    f = W.shape[1]
    grid = (n // block_m,)
    return pl.pallas_call(
        _mm_kernel,
        grid=grid,
        in_specs=[
            pl.BlockSpec((block_m, k), lambda i: (i, 0)),
            pl.BlockSpec((k, f), lambda i: (0, 0)),
            pl.BlockSpec((f,), lambda i: (0,)),
        ],
        out_specs=pl.BlockSpec((block_m, f), lambda i: (i, 0)),
        out_shape=jax.ShapeDtypeStruct((n, f), jnp.float32),
    )(x, W, b)


def _finish_mm_kernel(num_ref, den_ref, bias_ref, wl_ref, bl_ref, wr_ref,
                      br_ref, xl_ref, xr_ref, *, heads, hd):
    num = num_ref[...]
    den = den_ref[...]
    if heads > 1:
        den = jnp.repeat(den, hd, axis=1)
    x = num / (den + 1e-16) + bias_ref[...]
    x = jnp.where(x >= 0, x, 0.2 * x)
    xl_ref[...] = jnp.dot(x, wl_ref[...],
                          preferred_element_type=jnp.float32) + bl_ref[...]
    xr_ref[...] = jnp.dot(x, wr_ref[...],
                          preferred_element_type=jnp.float32) + br_ref[...]


def _finish_and_transform(num, den, bias, Wl, bl, Wr, br, heads, hd,
                          block_m=4000):
    n, f = num.shape
    fo = Wl.shape[1]
    kern = functools.partial(_finish_mm_kernel, heads=heads, hd=hd)
    grid = (n // block_m,)
    xl, xr = pl.pallas_call(
        kern,
        grid=grid,
        in_specs=[
            pl.BlockSpec((block_m, f), lambda i: (i, 0)),
            pl.BlockSpec((block_m, heads), lambda i: (i, 0)),
            pl.BlockSpec((f,), lambda i: (0,)),
            pl.BlockSpec((f, fo), lambda i: (0, 0)),
            pl.BlockSpec((fo,), lambda i: (0,)),
            pl.BlockSpec((f, fo), lambda i: (0, 0)),
            pl.BlockSpec((fo,), lambda i: (0,)),
        ],
        out_specs=[
            pl.BlockSpec((block_m, fo), lambda i: (i, 0)),
            pl.BlockSpec((block_m, fo), lambda i: (i, 0)),
        ],
        out_shape=[
            jax.ShapeDtypeStruct((n, fo), jnp.float32),
            jax.ShapeDtypeStruct((n, fo), jnp.float32),
        ],
    )(num, den, bias, Wl, bl, Wr, br)
    return xl, xr


def _finish_only_kernel(num_ref, den_ref, bias_ref, o_ref, *, heads, hd):
    num = num_ref[...]
    den = den_ref[...]
    if heads > 1:
        den = jnp.repeat(den, hd, axis=1)
    x = num / (den + 1e-16) + bias_ref[...]
    o_ref[...] = jnp.where(x >= 0, x, 0.2 * x)


def _finish_only(num, den, bias, heads, hd, block_m=4000):
    n, f = num.shape
    kern = functools.partial(_finish_only_kernel, heads=heads, hd=hd)
    return pl.pallas_call(
        kern,
        grid=(n // block_m,),
        in_specs=[
            pl.BlockSpec((block_m, f), lambda i: (i, 0)),
            pl.BlockSpec((block_m, heads), lambda i: (i, 0)),
            pl.BlockSpec((f,), lambda i: (0,)),
        ],
        out_specs=pl.BlockSpec((block_m, f), lambda i: (i, 0)),
        out_shape=jax.ShapeDtypeStruct((n, f), jnp.float32),
    )(num, den, bias)


def _fc_head_kernel(s_ref, a_ref, wsb_ref, wab_ref, fcw_ref, fcb_ref,
                    fow_ref, fob_ref, o_ref, acc):
    kb = pl.program_id(0)
    nk = pl.num_programs(0)

    @pl.when(kb == 0)
    def _init():
        acc[...] = jnp.zeros_like(acc)

    acc[...] += (
        jnp.dot(s_ref[...], wsb_ref[...], preferred_element_type=jnp.float32)
        + jnp.dot(a_ref[...], wab_ref[...], preferred_element_type=jnp.float32)
    )

    @pl.when(kb == nk - 1)
    def _fin():
        h = acc[...] + fcb_ref[...]
        q = jnp.maximum(h, 0.0)
        q = jnp.dot(q, fow_ref[...], preferred_element_type=jnp.float32) \
            + fob_ref[...]
        o_ref[...] = q


def _fc_head(s, a, fc_s_W, fc_s_b, fc_a_W, fc_a_b, fc_cat_W, fc_cat_b,
             fc_out_W, fc_out_b):
    KS = s.shape[1]
    KA = a.shape[1]
    BK = 640
    nks = KS // BK
    h12 = pl.pallas_call(
        functools.partial(_fc_stage1_kernel, nka=KA // BK),
        grid=(nks,),
        in_specs=[
            pl.BlockSpec((B, BK), lambda i: (0, i)),
            pl.BlockSpec((B, BK), lambda i: (0, jnp.minimum(i, KA // BK - 1))),
            pl.BlockSpec((BK, H1), lambda i: (i, 0)),
            pl.BlockSpec((BK, H1), lambda i: (jnp.minimum(i, KA // BK - 1), 0)),
        ],
        out_specs=pl.BlockSpec((B, 2 * H1), lambda i: (0, 0)),
        out_shape=jax.ShapeDtypeStruct((B, 2 * H1), jnp.float32),
        scratch_shapes=[pltpu.VMEM((B, 2 * H1), jnp.float32)],
    )(s, a, fc_s_W, fc_a_W)
    h12 = h12 + jnp.concatenate([fc_s_b, fc_a_b])[None, :]
    q = pl.pallas_call(
        _fc_stage2_kernel,
        in_specs=[
            pl.BlockSpec((B, 2 * H1), lambda: (0, 0)),
            pl.BlockSpec((2 * H1, H1), lambda: (0, 0)),
            pl.BlockSpec((H1,), lambda: (0,)),
            pl.BlockSpec((H1, 1), lambda: (0, 0)),
            pl.BlockSpec((1,), lambda: (0,)),
        ],
        out_specs=pl.BlockSpec((B, 1), lambda: (0, 0)),
        out_shape=jax.ShapeDtypeStruct((B, 1), jnp.float32),
    )(h12, fc_cat_W, fc_cat_b, fc_out_W, fc_out_b)
    return q


def _fc_stage1_kernel(s_ref, a_ref, ws_ref, wa_ref, o_ref, acc, *, nka):
    i = pl.program_id(0)
    nks = pl.num_programs(0)

    @pl.when(i == 0)
    def _init():
        acc[...] = jnp.zeros_like(acc)

    h1 = jnp.dot(s_ref[...], ws_ref[...], preferred_element_type=jnp.float32)
    acc[:, :H1] += h1

    @pl.when(i < nka)
    def _a_part():
        h2 = jnp.dot(a_ref[...], wa_ref[...],
                     preferred_element_type=jnp.float32)
        acc[:, H1:] += h2

    @pl.when(i == nks - 1)
    def _fin():
        o_ref[...] = acc[...]


def _fc_stage2_kernel(h_ref, wc_ref, bc_ref, wo_ref, bo_ref, o_ref):
    q = jnp.dot(h_ref[...], wc_ref[...], preferred_element_type=jnp.float32) \
        + bc_ref[...]
    q = jnp.maximum(q, 0.0)
    o_ref[...] = jnp.dot(q, wo_ref[...],
                         preferred_element_type=jnp.float32) + bo_ref[...]


def _edge_pass(xl, xr, src, dst, att, heads, hd):
    f = heads * hd
    z = xl[src] + xr[dst]
    z = jnp.where(z >= 0, z, 0.2 * z)
    logit = (z.reshape(EP, heads, hd) * att[None, :, :]).sum(-1)
    ex = jnp.exp(logit)
    den = jax.ops.segment_sum(ex, dst, num_segments=N)
    contrib = xl[src].reshape(EP, heads, hd) * ex[:, :, None]
    num = jax.ops.segment_sum(contrib.reshape(EP, f), dst, num_segments=N)
    return num, den


def kernel(g_x, next_demand, a, edge_index, reverse_edge_index,
           Wl3, bl3, Wr3, br3, att3, bias3,
           Wl4, bl4, Wr4, br4, att4, bias4,
           Wl1, bl1, Wr1, br1, att1, bias1,
           Wl2, bl2, Wr2, br2, att2, bias2,
           fc_s_W, fc_s_b, fc_a_W, fc_a_b, fc_cat_W, fc_cat_b,
           fc_out_W, fc_out_b):
    loop = jnp.arange(N, dtype=edge_index.dtype)
    src_r = jnp.concatenate([edge_index[1], loop])
    dst_r = jnp.concatenate([edge_index[0], loop])
    src_f = jnp.concatenate([edge_index[0], loop])
    dst_f = jnp.concatenate([edge_index[1], loop])

    x0 = jnp.concatenate([g_x, next_demand], axis=1)

    xl3 = _node_mm(x0, Wl3, bl3)
    xr3 = _node_mm(x0, Wr3, br3)
    num3, den3 = _edge_pass(xl3, xr3, src_r, dst_r, att3, HEADS, HID)

    xl4, xr4 = _finish_and_transform(num3, den3, bias3, Wl4, bl4, Wr4, br4,
                                     HEADS, HID)
    num4, den4 = _edge_pass(xl4, xr4, src_r, dst_r, att4, 1, OUT_CH)

    xl1, xr1 = _finish_and_transform(num4, den4, bias4, Wl1, bl1, Wr1, br1,
                                     1, OUT_CH)
    ub = _finish_only(num4, den4, bias4, 1, OUT_CH)
    num1, den1 = _edge_pass(xl1, xr1, src_f, dst_f, att1, HEADS, HID)

    xl2, xr2 = _finish_and_transform(num1, den1, bias1, Wl2, bl2, Wr2, br2,
                                     HEADS, HID)
    num2, den2 = _edge_pass(xl2, xr2, src_f, dst_f, att2, 1, OUT_CH)
    uf = _finish_only(num2, den2, bias2, 1, OUT_CH)

    s = jnp.concatenate([uf, ub], axis=1).reshape(B, 2 * OUT_CH * N_NODE)
    a_graph = a.reshape(B, N_GOODS * N_EDGE)
    return _fc_head(s, a_graph, fc_s_W, fc_s_b, fc_a_W, fc_a_b,
                    fc_cat_W, fc_cat_b, fc_out_W, fc_out_b)


# speedup vs baseline: 13.2637x; 1.7045x over previous
import functools

import jax
import jax.numpy as jnp
from jax import lax
from jax.experimental import pallas as pl
from jax.experimental.pallas import tpu as pltpu
from jax.experimental.pallas import tpu_sc as plsc

N_NODE = 1000
N_EDGE = 4000
N_GOODS = 4
B = 64
N = B * N_NODE
E = B * N_EDGE
IN_CH = 16
HID = 32
HEADS = 4
OUT_CH = 16
H1 = 1024
EP = E + N


def _mm_kernel(x_ref, w_ref, b_ref, o_ref):
    o_ref[...] = jnp.dot(x_ref[...], w_ref[...],
                         preferred_element_type=jnp.float32) + b_ref[...]


def _node_mm(x, W, b, block_m=2000):
    n, k = x.shape
    f = W.shape[1]
    grid = (n // block_m,)
    return pl.pallas_call(
        _mm_kernel,
        grid=grid,
        in_specs=[
            pl.BlockSpec((block_m, k), lambda i: (i, 0)),
            pl.BlockSpec((k, f), lambda i: (0, 0)),
            pl.BlockSpec((f,), lambda i: (0,)),
        ],
        out_specs=pl.BlockSpec((block_m, f), lambda i: (i, 0)),
        out_shape=jax.ShapeDtypeStruct((n, f), jnp.float32),
    )(x, W, b)


def _finish_x(osc_refs, bias_ref, *, heads, hd):
    parts = [r[:, :16] + r[:, 16:] for r in osc_refs]
    num = parts[0] if len(parts) == 2 else jnp.concatenate(parts[:-1], axis=1)
    den = parts[-1][:, :heads]
    if heads > 1:
        den = jnp.repeat(den, hd, axis=1)
    x = num / (den + 1e-16) + bias_ref[...]
    return jnp.where(x >= 0, x, 0.2 * x)


def _finish_mm_kernel(*refs, heads, hd, nin):
    osc_refs = refs[:nin]
    bias_ref, wl_ref, bl_ref, wr_ref, br_ref, xl_ref, xr_ref = refs[nin:]
    x = _finish_x(osc_refs, bias_ref, heads=heads, hd=hd)
    xl_ref[...] = jnp.dot(x, wl_ref[...],
                          preferred_element_type=jnp.float32) + bl_ref[...]
    xr_ref[...] = jnp.dot(x, wr_ref[...],
                          preferred_element_type=jnp.float32) + br_ref[...]


def _finish_and_transform(osc, bias, Wl, bl, Wr, br, heads, hd,
                          block_m=2000):
    n = osc[0].shape[0]
    f = heads * hd
    fo = Wl.shape[1]
    kern = functools.partial(_finish_mm_kernel, heads=heads, hd=hd,
                             nin=len(osc))
    grid = (n // block_m,)
    xl, xr = pl.pallas_call(
        kern,
        grid=grid,
        in_specs=[pl.BlockSpec((block_m, 32), lambda i: (i, 0))
                  for _ in osc] + [
            pl.BlockSpec((f,), lambda i: (0,)),
            pl.BlockSpec((f, fo), lambda i: (0, 0)),
            pl.BlockSpec((fo,), lambda i: (0,)),
            pl.BlockSpec((f, fo), lambda i: (0, 0)),
            pl.BlockSpec((fo,), lambda i: (0,)),
        ],
        out_specs=[
            pl.BlockSpec((block_m, fo), lambda i: (i, 0)),
            pl.BlockSpec((block_m, fo), lambda i: (i, 0)),
        ],
        out_shape=[
            jax.ShapeDtypeStruct((n, fo), jnp.float32),
            jax.ShapeDtypeStruct((n, fo), jnp.float32),
        ],
    )(*osc, bias, Wl, bl, Wr, br)
    return xl, xr


def _finish_only_kernel(*refs, heads, hd, nin):
    osc_refs = refs[:nin]
    bias_ref, o_ref = refs[nin:]
    o_ref[...] = _finish_x(osc_refs, bias_ref, heads=heads, hd=hd)


def _finish_only(osc, bias, heads, hd, block_m=2000):
    n = osc[0].shape[0]
    f = heads * hd
    kern = functools.partial(_finish_only_kernel, heads=heads, hd=hd,
                             nin=len(osc))
    return pl.pallas_call(
        kern,
        grid=(n // block_m,),
        in_specs=[pl.BlockSpec((block_m, 32), lambda i: (i, 0))
                  for _ in osc] + [
            pl.BlockSpec((f,), lambda i: (0,)),
        ],
        out_specs=pl.BlockSpec((block_m, f), lambda i: (i, 0)),
        out_shape=jax.ShapeDtypeStruct((n, f), jnp.float32),
    )(*osc, bias)


def _fc_head_kernel(s_ref, a_ref, wsb_ref, wab_ref, fcw_ref, fcb_ref,
                    fow_ref, fob_ref, o_ref, acc):
    kb = pl.program_id(0)
    nk = pl.num_programs(0)

    @pl.when(kb == 0)
    def _init():
        acc[...] = jnp.zeros_like(acc)

    acc[...] += (
        jnp.dot(s_ref[...], wsb_ref[...], preferred_element_type=jnp.float32)
        + jnp.dot(a_ref[...], wab_ref[...], preferred_element_type=jnp.float32)
    )

    @pl.when(kb == nk - 1)
    def _fin():
        h = acc[...] + fcb_ref[...]
        q = jnp.maximum(h, 0.0)
        q = jnp.dot(q, fow_ref[...], preferred_element_type=jnp.float32) \
            + fob_ref[...]
        o_ref[...] = q


def _fc_head(s, a, fc_s_W, fc_s_b, fc_a_W, fc_a_b, fc_cat_W, fc_cat_b,
             fc_out_W, fc_out_b):
    KS = s.shape[1]
    KA = a.shape[1]
    BK = 640
    nks = KS // BK
    h12 = pl.pallas_call(
        functools.partial(_fc_stage1_kernel, nka=KA // BK),
        grid=(nks,),
        in_specs=[
            pl.BlockSpec((B, BK), lambda i: (0, i)),
            pl.BlockSpec((B, BK), lambda i: (0, jnp.minimum(i, KA // BK - 1))),
            pl.BlockSpec((BK, H1), lambda i: (i, 0)),
            pl.BlockSpec((BK, H1), lambda i: (jnp.minimum(i, KA // BK - 1), 0)),
        ],
        out_specs=pl.BlockSpec((B, 2 * H1), lambda i: (0, 0)),
        out_shape=jax.ShapeDtypeStruct((B, 2 * H1), jnp.float32),
        scratch_shapes=[pltpu.VMEM((B, 2 * H1), jnp.float32)],
    )(s, a, fc_s_W, fc_a_W)
    h12 = h12 + jnp.concatenate([fc_s_b, fc_a_b])[None, :]
    q = pl.pallas_call(
        _fc_stage2_kernel,
        in_specs=[
            pl.BlockSpec((B, 2 * H1), lambda: (0, 0)),
            pl.BlockSpec((2 * H1, H1), lambda: (0, 0)),
            pl.BlockSpec((H1,), lambda: (0,)),
            pl.BlockSpec((H1, 1), lambda: (0, 0)),
            pl.BlockSpec((1,), lambda: (0,)),
        ],
        out_specs=pl.BlockSpec((B, 1), lambda: (0, 0)),
        out_shape=jax.ShapeDtypeStruct((B, 1), jnp.float32),
    )(h12, fc_cat_W, fc_cat_b, fc_out_W, fc_out_b)
    return q


def _fc_stage1_kernel(s_ref, a_ref, ws_ref, wa_ref, o_ref, acc, *, nka):
    i = pl.program_id(0)
    nks = pl.num_programs(0)

    @pl.when(i == 0)
    def _init():
        acc[...] = jnp.zeros_like(acc)

    h1 = jnp.dot(s_ref[...], ws_ref[...], preferred_element_type=jnp.float32)
    acc[:, :H1] += h1

    @pl.when(i < nka)
    def _a_part():
        h2 = jnp.dot(a_ref[...], wa_ref[...],
                     preferred_element_type=jnp.float32)
        acc[:, H1:] += h2

    @pl.when(i == nks - 1)
    def _fin():
        o_ref[...] = acc[...]


def _fc_stage2_kernel(h_ref, wc_ref, bc_ref, wo_ref, bo_ref, o_ref):
    q = jnp.dot(h_ref[...], wc_ref[...], preferred_element_type=jnp.float32) \
        + bc_ref[...]
    q = jnp.maximum(q, 0.0)
    o_ref[...] = jnp.dot(q, wo_ref[...],
                         preferred_element_type=jnp.float32) + bo_ref[...]


_C = 128
_NCH = EP // _C
_NW = 32
_RPT = N // 16
_CPT = ((_NCH + _NW - 1) // _NW) * _C
_ZB = 200


def _gather_sc_body(xl2d, xr2d, src_h, dst_h, xg_h, yg_h,
                    srcv, dstv, xrows, yrows, sem, sem2):
    cid = lax.axis_index("c")
    sid = lax.axis_index("s")
    wid = cid * 16 + sid
    clo = (wid * _NCH) // _NW
    chi = ((wid + 1) * _NCH) // _NW

    @pl.loop(0, chi - clo)
    def _chunk(k):
        off = (clo + k) * _C
        pltpu.sync_copy(src_h.at[pl.ds(off, _C)], srcv)
        pltpu.sync_copy(dst_h.at[pl.ds(off, _C)], dstv)
        ca = pltpu.async_copy(xl2d.at[srcv], xrows, sem)
        cb = pltpu.async_copy(xr2d.at[dstv], yrows, sem2)
        ca.wait()
        cb.wait()
        pltpu.sync_copy(xrows, xg_h.at[pl.ds(off, _C)])
        pltpu.sync_copy(yrows, yg_h.at[pl.ds(off, _C)])


def _gather_pass(xl, xr, src, dst):
    f = xl.shape[1]
    mesh = plsc.VectorSubcoreMesh(core_axis_name="c", subcore_axis_name="s",
                                  num_cores=2, num_subcores=16)
    kern = pl.kernel(
        _gather_sc_body,
        out_type=[jax.ShapeDtypeStruct((EP, f), jnp.float32),
                  jax.ShapeDtypeStruct((EP, f), jnp.float32)],
        mesh=mesh,
        scratch_types=[
            pltpu.VMEM((_C,), jnp.int32),
            pltpu.VMEM((_C,), jnp.int32),
            pltpu.VMEM((_C, f), jnp.float32),
            pltpu.VMEM((_C, f), jnp.float32),
            pltpu.SemaphoreType.DMA,
            pltpu.SemaphoreType.DMA,
        ],
    )
    return kern(xl, xr, src, dst)


def _edge_sc_body(xg_h, yg_h, dst_h, att_h, *rest,
                  heads, nc):
    nout = nc + 1
    outs = rest[:nout]
    (acc, exhbm, dstv, xlrows, xrrows, xlc, exst, exst1,
     valb, valb2, attv, zb, ob) = rest[nout:]
    f = 128
    nt = 8
    hd = 128 // heads
    cid = lax.axis_index("c")
    sid = lax.axis_index("s")
    wid = cid * 16 + sid
    clo = (wid * _NCH) // _NW
    chi = ((wid + 1) * _NCH) // _NW
    row0 = sid * _RPT

    pltpu.sync_copy(att_h, attv)

    @pl.loop(0, _ZB)
    def _z0(i):
        zb[i, :] = jnp.zeros((16,), jnp.float32)

    def _zero_acc():
        @pl.loop(0, _RPT, step=_ZB)
        def _za(r):
            pltpu.sync_copy(zb, acc.at[pl.ds(row0 + r, _ZB)])

    def _writeout(p):
        @pl.loop(0, _RPT, step=_ZB)
        def _wo(r):
            pltpu.sync_copy(acc.at[pl.ds(row0 + r, _ZB)], ob)
            pltpu.sync_copy(
                ob, outs[p].at[pl.ds(row0 + r, _ZB), pl.ds(cid * 16, 16)])

    _zero_acc()
    plsc.subcore_barrier()

    attvv = [attv[pl.ds(t * 16, 16)] for t in range(nt)]
    iota = lax.iota(jnp.int32, 16)
    onehot = [(iota == t).astype(jnp.float32) for t in range(16)]

    @pl.loop(0, chi - clo)
    def _chunkA(k):
        off = (clo + k) * _C
        pltpu.sync_copy(dst_h.at[pl.ds(off, _C)], dstv)
        pltpu.sync_copy(xg_h.at[pl.ds(off, _C)], xlrows)
        pltpu.sync_copy(yg_h.at[pl.ds(off, _C)], xrrows)

        @pl.loop(0, 8)
        def _grp(g):
            lv = [jnp.zeros((16,), jnp.float32) for _ in range(heads)]
            for t in range(16):
                e = g * 16 + t
                terms = []
                for u in range(nt):
                    xc = xlrows[e, pl.ds(u * 16, 16)]
                    yc = xrrows[e, pl.ds(u * 16, 16)]
                    z = xc + yc
                    z = jnp.maximum(z, 0.2 * z)
                    terms.append(z * attvv[u])
                vph = nt // heads
                for h in range(heads):
                    sh = terms[h * vph]
                    for u in range(1, vph):
                        sh = sh + terms[h * vph + u]
                    lv[h] = lv[h] + jnp.sum(sh) * onehot[t]
            exv = [jnp.exp(lv[h]) for h in range(heads)]
            for h in range(heads):
                exst[h, pl.ds(g * 16, 16)] = exv[h]
            for t in range(16):
                e = g * 16 + t
                row = exv[0][t] * onehot[0]
                for h in range(1, heads):
                    row = row + exv[h][t] * onehot[h]
                valb[e, :] = row

        for hh in range(heads):
            pltpu.sync_copy(
                exst.at[hh], exhbm.at[wid, pl.ds(hh * _CPT + k * _C, _C)])
        pltpu.sync_copy(valb, acc.at[dstv], add=True)

    plsc.subcore_barrier()
    _writeout(nc)

    for c in range(nc):
        h = (c * 16) // hd
        _zero_acc()
        plsc.subcore_barrier()

        @pl.loop(0, chi - clo)
        def _chunkB(k):
            off = (clo + k) * _C
            pltpu.sync_copy(dst_h.at[pl.ds(off, _C)], dstv)
            pltpu.sync_copy(
                exhbm.at[wid, pl.ds(h * _CPT + k * _C, _C)], exst1)
            pltpu.sync_copy(
                xg_h.at[pl.ds(off, _C), pl.ds(c * 16, 16)], xlc)

            @pl.loop(0, 8)
            def _scale(g):
                exv = exst1[pl.ds(g * 16, 16)]
                for t in range(16):
                    e = g * 16 + t
                    valb2[e, :] = xlc[e, :] * exv[t]

            pltpu.sync_copy(valb2, acc.at[dstv], add=True)

        plsc.subcore_barrier()
        _writeout(c)


def _edge_pass(xl, xr, src, dst, att, heads, hd):
    xg, yg = _gather_pass(xl, xr, src, dst)
    attf = att.reshape(heads * hd)
    if heads * hd < 128:
        attf = jnp.pad(attf, (0, 128 - heads * hd))
    return _edge_scatter(xg, yg, dst, attf, heads, heads * hd // 16)


def _edge_scatter(xg, yg, dst, attf, heads, nc):
    f = 128
    cpt = ((_NCH + _NW - 1) // _NW) * _C
    mesh = plsc.VectorSubcoreMesh(core_axis_name="c", subcore_axis_name="s",
                                  num_cores=2, num_subcores=16)
    kern = pl.kernel(
        functools.partial(_edge_sc_body, heads=heads, nc=nc),
        out_type=[jax.ShapeDtypeStruct((N, 32), jnp.float32)
                  for _ in range(nc + 1)],
        mesh=mesh,
        compiler_params=pltpu.CompilerParams(use_tc_tiling_on_sc=False, needs_layout_passes=False),
        scratch_types=[
            pltpu.VMEM_SHARED((N, 16), jnp.float32),
            pltpu.HBM((_NW, heads * _CPT), jnp.float32),
            pltpu.VMEM((_C,), jnp.int32),
            pltpu.VMEM((_C, f), jnp.float32),
            pltpu.VMEM((_C, f), jnp.float32),
            pltpu.VMEM((_C, 16), jnp.float32),
            pltpu.VMEM((heads, _C), jnp.float32),
            pltpu.VMEM((_C,), jnp.float32),
            pltpu.VMEM((_C, 16), jnp.float32),
            pltpu.VMEM((_C, 16), jnp.float32),
            pltpu.VMEM((f,), jnp.float32),
            pltpu.VMEM((_ZB, 16), jnp.float32),
            pltpu.VMEM((_ZB, 16), jnp.float32),
        ],
    )
    return kern(xg, yg, dst, attf)


def kernel(g_x, next_demand, a, edge_index, reverse_edge_index,
           Wl3, bl3, Wr3, br3, att3, bias3,
           Wl4, bl4, Wr4, br4, att4, bias4,
           Wl1, bl1, Wr1, br1, att1, bias1,
           Wl2, bl2, Wr2, br2, att2, bias2,
           fc_s_W, fc_s_b, fc_a_W, fc_a_b, fc_cat_W, fc_cat_b,
           fc_out_W, fc_out_b):
    loop = jnp.arange(N, dtype=edge_index.dtype)
    src_r = jnp.concatenate([edge_index[1], loop])
    dst_r = jnp.concatenate([edge_index[0], loop])
    src_f = jnp.concatenate([edge_index[0], loop])
    dst_f = jnp.concatenate([edge_index[1], loop])

    x0 = jnp.concatenate([g_x, next_demand], axis=1)

    xl3 = _node_mm(x0, Wl3, bl3)
    xr3 = _node_mm(x0, Wr3, br3)
    osc3 = _edge_pass(xl3, xr3, src_r, dst_r, att3, HEADS, HID)

    pad = ((0, 0), (0, 128 - OUT_CH))
    xl4, xr4 = _finish_and_transform(
        osc3, bias3, jnp.pad(Wl4, pad), jnp.pad(bl4, pad[1]),
        jnp.pad(Wr4, pad), jnp.pad(br4, pad[1]), HEADS, HID)
    osc4 = _edge_pass(xl4, xr4, src_r, dst_r, att4, 1, OUT_CH)

    xl1, xr1 = _finish_and_transform(osc4, bias4, Wl1, bl1, Wr1, br1,
                                     1, OUT_CH)
    ub = _finish_only(osc4, bias4, 1, OUT_CH)
    osc1 = _edge_pass(xl1, xr1, src_f, dst_f, att1, HEADS, HID)

    xl2, xr2 = _finish_and_transform(
        osc1, bias1, jnp.pad(Wl2, pad), jnp.pad(bl2, pad[1]),
        jnp.pad(Wr2, pad), jnp.pad(br2, pad[1]), HEADS, HID)
    osc2 = _edge_pass(xl2, xr2, src_f, dst_f, att2, 1, OUT_CH)
    uf = _finish_only(osc2, bias2, 1, OUT_CH)

    s = jnp.concatenate([uf, ub], axis=1).reshape(B, 2 * OUT_CH * N_NODE)
    a_graph = a.reshape(B, N_GOODS * N_EDGE)
    return _fc_head(s, a_graph, fc_s_W, fc_s_b, fc_a_W, fc_a_b,
                    fc_cat_W, fc_cat_b, fc_out_W, fc_out_b)


# speedup vs baseline: 18.4066x; 1.3877x over previous
import functools

import jax
import jax.numpy as jnp
from jax import lax
from jax.experimental import pallas as pl
from jax.experimental.pallas import tpu as pltpu
from jax.experimental.pallas import tpu_sc as plsc

N_NODE = 1000
N_EDGE = 4000
N_GOODS = 4
B = 64
N = B * N_NODE
E = B * N_EDGE
IN_CH = 16
HID = 32
HEADS = 4
OUT_CH = 16
H1 = 1024
EP = E + N


def _mm_kernel(x_ref, w_ref, b_ref, o_ref):
    o_ref[...] = jnp.dot(x_ref[...], w_ref[...],
                         preferred_element_type=jnp.float32) + b_ref[...]


def _node_mm(x, W, b, block_m=2000):
    n, k = x.shape
    f = W.shape[1]
    grid = (n // block_m,)
    return pl.pallas_call(
        _mm_kernel,
        grid=grid,
        in_specs=[
            pl.BlockSpec((block_m, k), lambda i: (i, 0)),
            pl.BlockSpec((k, f), lambda i: (0, 0)),
            pl.BlockSpec((f,), lambda i: (0,)),
        ],
        out_specs=pl.BlockSpec((block_m, f), lambda i: (i, 0)),
        out_shape=jax.ShapeDtypeStruct((n, f), jnp.float32),
    )(x, W, b)


def _finish_x(osc_refs, bias_ref, *, heads, hd):
    parts = [r[:, :16] + r[:, 16:] for r in osc_refs]
    num = parts[0] if len(parts) == 2 else jnp.concatenate(parts[:-1], axis=1)
    den = parts[-1][:, :heads]
    if heads > 1:
        den = jnp.repeat(den, hd, axis=1)
    x = num / (den + 1e-16) + bias_ref[...]
    return jnp.where(x >= 0, x, 0.2 * x)


def _finish_mm_kernel(*refs, heads, hd, nin):
    osc_refs = refs[:nin]
    bias_ref, wl_ref, bl_ref, wr_ref, br_ref, xl_ref, xr_ref = refs[nin:]
    x = _finish_x(osc_refs, bias_ref, heads=heads, hd=hd)
    xl_ref[...] = jnp.dot(x, wl_ref[...],
                          preferred_element_type=jnp.float32) + bl_ref[...]
    xr_ref[...] = jnp.dot(x, wr_ref[...],
                          preferred_element_type=jnp.float32) + br_ref[...]


def _finish_and_transform(osc, bias, Wl, bl, Wr, br, heads, hd,
                          block_m=2000):
    n = osc[0].shape[0]
    f = heads * hd
    fo = Wl.shape[1]
    kern = functools.partial(_finish_mm_kernel, heads=heads, hd=hd,
                             nin=len(osc))
    grid = (n // block_m,)
    xl, xr = pl.pallas_call(
        kern,
        grid=grid,
        in_specs=[pl.BlockSpec((block_m, 32), lambda i: (i, 0))
                  for _ in osc] + [
            pl.BlockSpec((f,), lambda i: (0,)),
            pl.BlockSpec((f, fo), lambda i: (0, 0)),
            pl.BlockSpec((fo,), lambda i: (0,)),
            pl.BlockSpec((f, fo), lambda i: (0, 0)),
            pl.BlockSpec((fo,), lambda i: (0,)),
        ],
        out_specs=[
            pl.BlockSpec((block_m, fo), lambda i: (i, 0)),
            pl.BlockSpec((block_m, fo), lambda i: (i, 0)),
        ],
        out_shape=[
            jax.ShapeDtypeStruct((n, fo), jnp.float32),
            jax.ShapeDtypeStruct((n, fo), jnp.float32),
        ],
    )(*osc, bias, Wl, bl, Wr, br)
    return xl, xr


def _finish_only_kernel(*refs, heads, hd, nin):
    osc_refs = refs[:nin]
    bias_ref, o_ref = refs[nin:]
    o_ref[...] = _finish_x(osc_refs, bias_ref, heads=heads, hd=hd)


def _finish_only(osc, bias, heads, hd, block_m=2000):
    n = osc[0].shape[0]
    f = heads * hd
    kern = functools.partial(_finish_only_kernel, heads=heads, hd=hd,
                             nin=len(osc))
    return pl.pallas_call(
        kern,
        grid=(n // block_m,),
        in_specs=[pl.BlockSpec((block_m, 32), lambda i: (i, 0))
                  for _ in osc] + [
            pl.BlockSpec((f,), lambda i: (0,)),
        ],
        out_specs=pl.BlockSpec((block_m, f), lambda i: (i, 0)),
        out_shape=jax.ShapeDtypeStruct((n, f), jnp.float32),
    )(*osc, bias)


def _fc_head_kernel(s_ref, a_ref, wsb_ref, wab_ref, fcw_ref, fcb_ref,
                    fow_ref, fob_ref, o_ref, acc):
    kb = pl.program_id(0)
    nk = pl.num_programs(0)

    @pl.when(kb == 0)
    def _init():
        acc[...] = jnp.zeros_like(acc)

    acc[...] += (
        jnp.dot(s_ref[...], wsb_ref[...], preferred_element_type=jnp.float32)
        + jnp.dot(a_ref[...], wab_ref[...], preferred_element_type=jnp.float32)
    )

    @pl.when(kb == nk - 1)
    def _fin():
        h = acc[...] + fcb_ref[...]
        q = jnp.maximum(h, 0.0)
        q = jnp.dot(q, fow_ref[...], preferred_element_type=jnp.float32) \
            + fob_ref[...]
        o_ref[...] = q


def _fc_head(s, a, fc_s_W, fc_s_b, fc_a_W, fc_a_b, fc_cat_W, fc_cat_b,
             fc_out_W, fc_out_b):
    KS = s.shape[1]
    KA = a.shape[1]
    BK = 640
    nks = KS // BK
    h12 = pl.pallas_call(
        functools.partial(_fc_stage1_kernel, nka=KA // BK),
        grid=(nks,),
        in_specs=[
            pl.BlockSpec((B, BK), lambda i: (0, i)),
            pl.BlockSpec((B, BK), lambda i: (0, jnp.minimum(i, KA // BK - 1))),
            pl.BlockSpec((BK, H1), lambda i: (i, 0)),
            pl.BlockSpec((BK, H1), lambda i: (jnp.minimum(i, KA // BK - 1), 0)),
        ],
        out_specs=pl.BlockSpec((B, 2 * H1), lambda i: (0, 0)),
        out_shape=jax.ShapeDtypeStruct((B, 2 * H1), jnp.float32),
        scratch_shapes=[pltpu.VMEM((B, 2 * H1), jnp.float32)],
    )(s, a, fc_s_W, fc_a_W)
    h12 = h12 + jnp.concatenate([fc_s_b, fc_a_b])[None, :]
    q = pl.pallas_call(
        _fc_stage2_kernel,
        in_specs=[
            pl.BlockSpec((B, 2 * H1), lambda: (0, 0)),
            pl.BlockSpec((2 * H1, H1), lambda: (0, 0)),
            pl.BlockSpec((H1,), lambda: (0,)),
            pl.BlockSpec((H1, 1), lambda: (0, 0)),
            pl.BlockSpec((1,), lambda: (0,)),
        ],
        out_specs=pl.BlockSpec((B, 1), lambda: (0, 0)),
        out_shape=jax.ShapeDtypeStruct((B, 1), jnp.float32),
    )(h12, fc_cat_W, fc_cat_b, fc_out_W, fc_out_b)
    return q


def _fc_stage1_kernel(s_ref, a_ref, ws_ref, wa_ref, o_ref, acc, *, nka):
    i = pl.program_id(0)
    nks = pl.num_programs(0)

    @pl.when(i == 0)
    def _init():
        acc[...] = jnp.zeros_like(acc)

    h1 = jnp.dot(s_ref[...], ws_ref[...], preferred_element_type=jnp.float32)
    acc[:, :H1] += h1

    @pl.when(i < nka)
    def _a_part():
        h2 = jnp.dot(a_ref[...], wa_ref[...],
                     preferred_element_type=jnp.float32)
        acc[:, H1:] += h2

    @pl.when(i == nks - 1)
    def _fin():
        o_ref[...] = acc[...]


def _fc_stage2_kernel(h_ref, wc_ref, bc_ref, wo_ref, bo_ref, o_ref):
    q = jnp.dot(h_ref[...], wc_ref[...], preferred_element_type=jnp.float32) \
        + bc_ref[...]
    q = jnp.maximum(q, 0.0)
    o_ref[...] = jnp.dot(q, wo_ref[...],
                         preferred_element_type=jnp.float32) + bo_ref[...]


_C = 128
_NCH = EP // _C
_NW = 32
_RPT = N // 16
_CPT = ((_NCH + _NW - 1) // _NW) * _C
_ZB = 200


def _gather_sc_body(xl2d, xr2d, src_h, dst_h, xg_h, yg_h,
                    srcall, dstall, xrows, yrows, sem, sem2):
    cid = lax.axis_index("c")
    sid = lax.axis_index("s")
    wid = cid * 16 + sid
    clo = (wid * _NCH) // _NW
    chi = ((wid + 1) * _NCH) // _NW

    pltpu.sync_copy(src_h.at[pl.ds(clo * _C, _CPT)], srcall)
    pltpu.sync_copy(dst_h.at[pl.ds(clo * _C, _CPT)], dstall)

    @pl.loop(0, chi - clo)
    def _chunk(k):
        off = (clo + k) * _C
        ca = pltpu.async_copy(
            xl2d.at[srcall.at[pl.ds(k * _C, _C)]], xrows, sem)
        cb = pltpu.async_copy(
            xr2d.at[dstall.at[pl.ds(k * _C, _C)]], yrows, sem2)
        ca.wait()
        cb.wait()
        cc = pltpu.async_copy(xrows, xg_h.at[pl.ds(off, _C)], sem)
        cd = pltpu.async_copy(yrows, yg_h.at[pl.ds(off, _C)], sem2)
        cc.wait()
        cd.wait()


def _gather_pass(xl, xr, src, dst):
    f = xl.shape[1]
    mesh = plsc.VectorSubcoreMesh(core_axis_name="c", subcore_axis_name="s",
                                  num_cores=2, num_subcores=16)
    kern = pl.kernel(
        _gather_sc_body,
        out_type=[jax.ShapeDtypeStruct((EP, f), jnp.float32),
                  jax.ShapeDtypeStruct((EP, f), jnp.float32)],
        mesh=mesh,
        scratch_types=[
            pltpu.VMEM((_CPT,), jnp.int32),
            pltpu.VMEM((_CPT,), jnp.int32),
            pltpu.VMEM((_C, f), jnp.float32),
            pltpu.VMEM((_C, f), jnp.float32),
            pltpu.SemaphoreType.DMA,
            pltpu.SemaphoreType.DMA,
        ],
    )
    return kern(xl, xr, src, dst)


def _edge_sc_body(xg_h, yg_h, dst_h, att_h, *rest, heads, nc):
    nout = nc + 1
    outs = rest[:nout]
    (acc, exhbm, dstall, dstv, xlrows, xrrows, xlc, exst1d, exst1,
     valb, valb2, attv, zb, ob, sem, sem2) = rest[nout:]
    f = 128
    nt = 8
    hd = 128 // heads
    hc = heads * _C
    cid = lax.axis_index("c")
    sid = lax.axis_index("s")
    wid = cid * 16 + sid
    clo = (wid * _NCH) // _NW
    chi = ((wid + 1) * _NCH) // _NW
    row0 = sid * _RPT

    pltpu.sync_copy(att_h, attv)
    pltpu.sync_copy(dst_h.at[pl.ds(clo * _C, _CPT)], dstall)

    @pl.loop(0, _ZB)
    def _z0(i):
        zb[i, :] = jnp.zeros((16,), jnp.float32)

    def _zero_acc():
        @pl.loop(0, _RPT, step=_ZB)
        def _za(r):
            pltpu.sync_copy(zb, acc.at[pl.ds(row0 + r, _ZB)])

    def _writeout(p):
        @pl.loop(0, _RPT, step=_ZB)
        def _wo(r):
            pltpu.sync_copy(acc.at[pl.ds(row0 + r, _ZB)], ob)
            pltpu.sync_copy(
                ob, outs[p].at[pl.ds(row0 + r, _ZB), pl.ds(cid * 16, 16)])

    def _fill_dstv(k):
        @pl.loop(0, 8)
        def _cp(g):
            dstv[pl.ds(g * 16, 16)] = dstall[pl.ds(k * _C + g * 16, 16)]

    _zero_acc()
    plsc.subcore_barrier()

    attvv = [attv[pl.ds(t * 16, 16)] for t in range(nt)]
    iota = lax.iota(jnp.int32, 16)
    onehot = [(iota == t).astype(jnp.float32) for t in range(16)]

    @pl.loop(0, chi - clo)
    def _chunkA(k):
        off = (clo + k) * _C
        ca = pltpu.async_copy(xg_h.at[pl.ds(off, _C)], xlrows, sem)
        cb = pltpu.async_copy(yg_h.at[pl.ds(off, _C)], xrrows, sem2)
        _fill_dstv(k)
        ca.wait()
        cb.wait()

        @pl.loop(0, 8)
        def _grp(g):
            lv = [jnp.zeros((16,), jnp.float32) for _ in range(heads)]
            for t in range(16):
                e = g * 16 + t
                terms = []
                for u in range(nt):
                    xc = xlrows[e, pl.ds(u * 16, 16)]
                    yc = xrrows[e, pl.ds(u * 16, 16)]
                    z = xc + yc
                    z = jnp.maximum(z, 0.2 * z)
                    terms.append(z * attvv[u])
                vph = nt // heads
                for h in range(heads):
                    sh = terms[h * vph]
                    for u in range(1, vph):
                        sh = sh + terms[h * vph + u]
                    lv[h] = lv[h] + jnp.sum(sh) * onehot[t]
            exv = [jnp.exp(lv[h]) for h in range(heads)]
            for h in range(heads):
                exst1d[pl.ds(h * _C + g * 16, 16)] = exv[h]
            for t in range(16):
                e = g * 16 + t
                row = exv[0][t] * onehot[0]
                for h in range(1, heads):
                    row = row + exv[h][t] * onehot[h]
                valb[e, :] = row

        cw = pltpu.async_copy(
            exst1d, exhbm.at[wid, pl.ds(k * hc, hc)], sem)
        pltpu.sync_copy(valb, acc.at[dstv], add=True)
        cw.wait()

    plsc.subcore_barrier()
    _writeout(nc)

    for c in range(nc):
        h = (c * 16) // hd
        _zero_acc()
        plsc.subcore_barrier()

        @pl.loop(0, chi - clo)
        def _chunkB(k):
            off = (clo + k) * _C
            ca = pltpu.async_copy(
                exhbm.at[wid, pl.ds(k * hc + h * _C, _C)], exst1, sem)
            cb = pltpu.async_copy(
                xg_h.at[pl.ds(off, _C), pl.ds(c * 16, 16)], xlc, sem2)
            _fill_dstv(k)
            ca.wait()
            cb.wait()

            @pl.loop(0, 8)
            def _scale(g):
                exv = exst1[pl.ds(g * 16, 16)]
                for t in range(16):
                    e = g * 16 + t
                    valb2[e, :] = xlc[e, :] * exv[t]

            pltpu.sync_copy(valb2, acc.at[dstv], add=True)

        plsc.subcore_barrier()
        _writeout(c)


def _edge_pass(xl, xr, src, dst, att, heads, hd):
    xg, yg = _gather_pass(xl, xr, src, dst)
    attf = att.reshape(heads * hd)
    if heads * hd < 128:
        attf = jnp.pad(attf, (0, 128 - heads * hd))
    return _edge_scatter(xg, yg, dst, attf, heads, heads * hd // 16)


def _edge_scatter(xg, yg, dst, attf, heads, nc):
    f = 128
    cpt = ((_NCH + _NW - 1) // _NW) * _C
    mesh = plsc.VectorSubcoreMesh(core_axis_name="c", subcore_axis_name="s",
                                  num_cores=2, num_subcores=16)
    kern = pl.kernel(
        functools.partial(_edge_sc_body, heads=heads, nc=nc),
        out_type=[jax.ShapeDtypeStruct((N, 32), jnp.float32)
                  for _ in range(nc + 1)],
        mesh=mesh,
        compiler_params=pltpu.CompilerParams(use_tc_tiling_on_sc=False, needs_layout_passes=False),
        scratch_types=[
            pltpu.VMEM_SHARED((N, 16), jnp.float32),
            pltpu.HBM((_NW, heads * _CPT), jnp.float32),
            pltpu.VMEM((_CPT,), jnp.int32),
            pltpu.VMEM((_C,), jnp.int32),
            pltpu.VMEM((_C, 128), jnp.float32),
            pltpu.VMEM((_C, 128), jnp.float32),
            pltpu.VMEM((_C, 16), jnp.float32),
            pltpu.VMEM((heads * _C,), jnp.float32),
            pltpu.VMEM((_C,), jnp.float32),
            pltpu.VMEM((_C, 16), jnp.float32),
            pltpu.VMEM((_C, 16), jnp.float32),
            pltpu.VMEM((128,), jnp.float32),
            pltpu.VMEM((_ZB, 16), jnp.float32),
            pltpu.VMEM((_ZB, 16), jnp.float32),
            pltpu.SemaphoreType.DMA,
            pltpu.SemaphoreType.DMA,
        ],
    )
    return kern(xg, yg, dst, attf)


def kernel(g_x, next_demand, a, edge_index, reverse_edge_index,
           Wl3, bl3, Wr3, br3, att3, bias3,
           Wl4, bl4, Wr4, br4, att4, bias4,
           Wl1, bl1, Wr1, br1, att1, bias1,
           Wl2, bl2, Wr2, br2, att2, bias2,
           fc_s_W, fc_s_b, fc_a_W, fc_a_b, fc_cat_W, fc_cat_b,
           fc_out_W, fc_out_b):
    loop = jnp.arange(N, dtype=edge_index.dtype)
    src_r = jnp.concatenate([edge_index[1], loop])
    dst_r = jnp.concatenate([edge_index[0], loop])
    src_f = jnp.concatenate([edge_index[0], loop])
    dst_f = jnp.concatenate([edge_index[1], loop])

    x0 = jnp.concatenate([g_x, next_demand], axis=1)

    xl3 = _node_mm(x0, Wl3, bl3)
    xr3 = _node_mm(x0, Wr3, br3)
    osc3 = _edge_pass(xl3, xr3, src_r, dst_r, att3, HEADS, HID)

    pad = ((0, 0), (0, 128 - OUT_CH))
    xl4, xr4 = _finish_and_transform(
        osc3, bias3, jnp.pad(Wl4, pad), jnp.pad(bl4, pad[1]),
        jnp.pad(Wr4, pad), jnp.pad(br4, pad[1]), HEADS, HID)
    osc4 = _edge_pass(xl4, xr4, src_r, dst_r, att4, 1, OUT_CH)

    xl1, xr1 = _finish_and_transform(osc4, bias4, Wl1, bl1, Wr1, br1,
                                     1, OUT_CH)
    ub = _finish_only(osc4, bias4, 1, OUT_CH)
    osc1 = _edge_pass(xl1, xr1, src_f, dst_f, att1, HEADS, HID)

    xl2, xr2 = _finish_and_transform(
        osc1, bias1, jnp.pad(Wl2, pad), jnp.pad(bl2, pad[1]),
        jnp.pad(Wr2, pad), jnp.pad(br2, pad[1]), HEADS, HID)
    osc2 = _edge_pass(xl2, xr2, src_f, dst_f, att2, 1, OUT_CH)
    uf = _finish_only(osc2, bias2, 1, OUT_CH)

    s = jnp.concatenate([uf, ub], axis=1).reshape(B, 2 * OUT_CH * N_NODE)
    a_graph = a.reshape(B, N_GOODS * N_EDGE)
    return _fc_head(s, a_graph, fc_s_W, fc_s_b, fc_a_W, fc_a_b,
                    fc_cat_W, fc_cat_b, fc_out_W, fc_out_b)
